# Initial kernel scaffold; baseline (speedup 1.0000x reference)
import jax
import jax.numpy as jnp
from jax.experimental import pallas as pl


def kernel(x, edge_index, node_type, enc_W1, enc_b1, enc_W2, enc_b2, gcn_W1, gcn_b1, gcn_W2, gcn_b2):
    raise NotImplementedError("write your pallas kernel here")


# speedup vs baseline: 24.8750x; 24.8750x over previous
import functools

import jax
import jax.numpy as jnp
from jax import lax
from jax.experimental import pallas as pl
from jax.experimental.pallas import tpu as pltpu
from jax.experimental.pallas import tpu_sc as plsc

N = 100000
E = 1600000
D_IN = 128
H_ENC = 64
D_ENC = 32
NT = 2

NP = 100096
EPR = 12544
EP = EPR * 128
R = 8
TILE_ROWS_SEG = EPR // 16
NBLK_SEG = TILE_ROWS_SEG // R
TILE_ROWS_DEG = EPR // 32
NBLK_DEG = TILE_ROWS_DEG // R
NPT = NP // 16

_mesh = plsc.VectorSubcoreMesh(core_axis_name="c", subcore_axis_name="s")

f32 = jnp.float32
i32 = jnp.int32


@functools.partial(
    pl.kernel,
    out_type=[jax.ShapeDtypeStruct((NP,), f32),
              jax.ShapeDtypeStruct((NP,), f32)],
    mesh=_mesh,
    compiler_params=pltpu.CompilerParams(use_tc_tiling_on_sc=False),
    scratch_types=[
        pltpu.VMEM((R, 128), i32),
        pltpu.VMEM((128,), f32),
        pltpu.VMEM((1024,), f32),
        pltpu.VMEM_SHARED((NP,), f32),
    ],
)
def _deg_kernel(dst2d, o0, o1, didx, ones_v, zb, acc):
    cid = lax.axis_index("c")
    sid = lax.axis_index("s")

    def fill_z(i, _):
        zb[pl.ds(i * 16, 16)] = jnp.zeros((16,), f32)
        return 0
    lax.fori_loop(0, 64, fill_z, 0)
    for j in range(8):
        ones_v[pl.ds(j * 16, 16)] = jnp.ones((16,), f32)

    t0 = sid * NPT
    for k in range(6):
        pltpu.sync_copy(zb, acc.at[pl.ds(t0 + k * 1024, 1024)])
    pltpu.sync_copy(zb.at[pl.ds(0, 112)], acc.at[pl.ds(t0 + 6144, 112)])
    plsc.subcore_barrier()

    w = cid * 16 + sid
    r_base = w * TILE_ROWS_DEG

    def blk(b, _):
        r0 = r_base + b * R
        pltpu.sync_copy(dst2d.at[pl.ds(r0, R)], didx)
        for j in range(R):
            pltpu.sync_copy(ones_v, acc.at[didx.at[j]], add=True)
        return 0
    lax.fori_loop(0, NBLK_DEG, blk, 0)
    plsc.subcore_barrier()

    def _writeback(out):
        for k in range(6):
            pltpu.sync_copy(acc.at[pl.ds(t0 + k * 1024, 1024)], zb)
            pltpu.sync_copy(zb, out.at[pl.ds(t0 + k * 1024, 1024)])
        pltpu.sync_copy(acc.at[pl.ds(t0 + 6144, 112)], zb.at[pl.ds(0, 112)])
        pltpu.sync_copy(zb.at[pl.ds(0, 112)], out.at[pl.ds(t0 + 6144, 112)])

    @pl.when(cid == 0)
    def _():
        _writeback(o0)

    @pl.when(cid == 1)
    def _():
        _writeback(o1)


@functools.partial(
    pl.kernel,
    out_type=[jax.ShapeDtypeStruct((NP, 16), f32),
              jax.ShapeDtypeStruct((NP, 16), f32)],
    mesh=_mesh,
    compiler_params=pltpu.CompilerParams(use_tc_tiling_on_sc=False),
    scratch_types=[
        pltpu.VMEM((R, 128), i32),
        pltpu.VMEM((R, 128), i32),
        pltpu.VMEM((R * 128, 16), f32),
        pltpu.VMEM_SHARED((NP, 16), f32),
        pltpu.SemaphoreType.DMA,
    ],
)
def _seg_kernel(ua, ub, src2d, dst2d, oa, ob, sidx, didx, rows, acc, sem):
    cid = lax.axis_index("c")
    sid = lax.axis_index("s")
    t0 = sid * NPT

    def run(table, out):
        def fill_z(i, _):
            rows[i, :] = jnp.zeros((16,), f32)
            return 0
        lax.fori_loop(0, R * 128, fill_z, 0)
        for k in range(6):
            pltpu.sync_copy(rows.at[pl.ds(0, 1024)],
                            acc.at[pl.ds(t0 + k * 1024, 1024)])
        pltpu.sync_copy(rows.at[pl.ds(0, 112)],
                        acc.at[pl.ds(t0 + 6144, 112)])
        plsc.subcore_barrier()

        r_base = sid * TILE_ROWS_SEG

        def blk(b, _):
            r0 = r_base + b * R
            pltpu.sync_copy(src2d.at[pl.ds(r0, R)], sidx)
            pltpu.sync_copy(dst2d.at[pl.ds(r0, R)], didx)
            handles = [
                pltpu.async_copy(table.at[sidx.at[j]],
                                 rows.at[pl.ds(j * 128, 128)], sem)
                for j in range(R)
            ]
            for h in handles:
                h.wait()
            for j in range(R):
                pltpu.sync_copy(rows.at[pl.ds(j * 128, 128)],
                                acc.at[didx.at[j]], add=True)
            return 0
        lax.fori_loop(0, NBLK_SEG, blk, 0)
        plsc.subcore_barrier()
        for k in range(6):
            pltpu.sync_copy(acc.at[pl.ds(t0 + k * 1024, 1024)],
                            rows.at[pl.ds(0, 1024)])
            pltpu.sync_copy(rows.at[pl.ds(0, 1024)],
                            out.at[pl.ds(t0 + k * 1024, 1024)])
        pltpu.sync_copy(acc.at[pl.ds(t0 + 6144, 112)],
                        rows.at[pl.ds(0, 112)])
        pltpu.sync_copy(rows.at[pl.ds(0, 112)],
                        out.at[pl.ds(t0 + 6144, 112)])

    @pl.when(cid == 0)
    def _():
        run(ua, oa)

    @pl.when(cid == 1)
    def _():
        run(ub, ob)


BN = 2000
GRID = N // BN


def _enc_prep_body(x_ref, nt_ref, d0_ref, d1_ref, w1_ref, b1_ref, w2_ref,
                   b2_ref, gw1_ref, ua_ref, ub_ref, dinv_ref):
    x = x_ref[...]
    za = jnp.maximum(jnp.dot(x, w1_ref[0], preferred_element_type=f32)
                     + b1_ref[0], 0.0)
    za = jnp.maximum(jnp.dot(za, w2_ref[0], preferred_element_type=f32)
                     + b2_ref[0], 0.0)
    zb = jnp.maximum(jnp.dot(x, w1_ref[1], preferred_element_type=f32)
                     + b1_ref[1], 0.0)
    zb = jnp.maximum(jnp.dot(zb, w2_ref[1], preferred_element_type=f32)
                     + b2_ref[1], 0.0)
    h = jnp.where(nt_ref[...] == 0, za, zb)
    dinv = lax.rsqrt(d0_ref[...] + d1_ref[...] + 1.0)
    u = jnp.dot(h, gw1_ref[...], preferred_element_type=f32) * dinv
    ua_ref[...] = u[:, :16]
    ub_ref[...] = u[:, 16:]
    dinv_ref[...] = dinv


def _enc_prep(x, nt2, d02, d12, enc_W1, enc_b1, enc_W2, enc_b2, gcn_W1):
    return pl.pallas_call(
        _enc_prep_body,
        grid=(GRID,),
        in_specs=[
            pl.BlockSpec((BN, D_IN), lambda i: (i, 0)),
            pl.BlockSpec((BN, 1), lambda i: (i, 0)),
            pl.BlockSpec((BN, 1), lambda i: (i, 0)),
            pl.BlockSpec((BN, 1), lambda i: (i, 0)),
            pl.BlockSpec((NT, D_IN, H_ENC), lambda i: (0, 0, 0)),
            pl.BlockSpec((NT, H_ENC), lambda i: (0, 0)),
            pl.BlockSpec((NT, H_ENC, D_ENC), lambda i: (0, 0, 0)),
            pl.BlockSpec((NT, D_ENC), lambda i: (0, 0)),
            pl.BlockSpec((D_ENC, D_ENC), lambda i: (0, 0)),
        ],
        out_specs=[
            pl.BlockSpec((BN, 16), lambda i: (i, 0)),
            pl.BlockSpec((BN, 16), lambda i: (i, 0)),
            pl.BlockSpec((BN, 1), lambda i: (i, 0)),
        ],
        out_shape=[
            jax.ShapeDtypeStruct((N, 16), f32),
            jax.ShapeDtypeStruct((N, 16), f32),
            jax.ShapeDtypeStruct((N, 1), f32),
        ],
    )(x, nt2, d02, d12, enc_W1, enc_b1, enc_W2, enc_b2, gcn_W1)


def _mid_body(sa_ref, sb_ref, ua_ref, ub_ref, dinv_ref, gw2_ref, b1_ref,
              h1_ref, va_ref, vb_ref):
    dinv = dinv_ref[...]
    t = jnp.concatenate([sa_ref[...] + ua_ref[...],
                         sb_ref[...] + ub_ref[...]], axis=1)
    h1 = jnp.maximum(t * dinv + b1_ref[...], 0.0)
    h1_ref[...] = h1
    v = jnp.dot(h1, gw2_ref[...], preferred_element_type=f32) * dinv
    va_ref[...] = v[:, :16]
    vb_ref[...] = v[:, 16:]


def _mid(s1a, s1b, u1a, u1b, dinv, gcn_W2, b1r):
    return pl.pallas_call(
        _mid_body,
        grid=(GRID,),
        in_specs=[
            pl.BlockSpec((BN, 16), lambda i: (i, 0)),
            pl.BlockSpec((BN, 16), lambda i: (i, 0)),
            pl.BlockSpec((BN, 16), lambda i: (i, 0)),
            pl.BlockSpec((BN, 16), lambda i: (i, 0)),
            pl.BlockSpec((BN, 1), lambda i: (i, 0)),
            pl.BlockSpec((D_ENC, D_ENC), lambda i: (0, 0)),
            pl.BlockSpec((1, D_ENC), lambda i: (0, 0)),
        ],
        out_specs=[
            pl.BlockSpec((BN, D_ENC), lambda i: (i, 0)),
            pl.BlockSpec((BN, 16), lambda i: (i, 0)),
            pl.BlockSpec((BN, 16), lambda i: (i, 0)),
        ],
        out_shape=[
            jax.ShapeDtypeStruct((N, D_ENC), f32),
            jax.ShapeDtypeStruct((N, 16), f32),
            jax.ShapeDtypeStruct((N, 16), f32),
        ],
    )(s1a, s1b, u1a, u1b, dinv, gcn_W2, b1r)


def _final_body(sa_ref, sb_ref, va_ref, vb_ref, dinv_ref, h1_ref, b2_ref,
                out_ref):
    dinv = dinv_ref[...]
    t = jnp.concatenate([sa_ref[...] + va_ref[...],
                         sb_ref[...] + vb_ref[...]], axis=1)
    h2 = jnp.maximum(t * dinv + b2_ref[...], 0.0)
    out_ref[...] = jnp.concatenate([h1_ref[...], h2], axis=1)


def _final(s2a, s2b, u2a, u2b, dinv, h1, b2r):
    return pl.pallas_call(
        _final_body,
        grid=(GRID,),
        in_specs=[
            pl.BlockSpec((BN, 16), lambda i: (i, 0)),
            pl.BlockSpec((BN, 16), lambda i: (i, 0)),
            pl.BlockSpec((BN, 16), lambda i: (i, 0)),
            pl.BlockSpec((BN, 16), lambda i: (i, 0)),
            pl.BlockSpec((BN, 1), lambda i: (i, 0)),
            pl.BlockSpec((BN, D_ENC), lambda i: (i, 0)),
            pl.BlockSpec((1, D_ENC), lambda i: (0, 0)),
        ],
        out_specs=pl.BlockSpec((BN, 2 * D_ENC), lambda i: (i, 0)),
        out_shape=jax.ShapeDtypeStruct((N, 2 * D_ENC), f32),
    )(s2a, s2b, u2a, u2b, dinv, h1, b2r)


def kernel(x, edge_index, node_type, enc_W1, enc_b1, enc_W2, enc_b2,
           gcn_W1, gcn_b1, gcn_W2, gcn_b2):
    pad = EP - E
    pad_src = (jnp.arange(pad, dtype=i32) % 128)
    pad_dst = N + (jnp.arange(pad, dtype=i32) % 16)
    src2d = jnp.concatenate([edge_index[0], pad_src]).reshape(EPR, 128)
    dst2d = jnp.concatenate([edge_index[1], pad_dst]).reshape(EPR, 128)

    d0, d1 = _deg_kernel(dst2d)
    d02 = d0[:, None]
    d12 = d1[:, None]

    nt2 = node_type[:, None]
    b1r = gcn_b1[None, :]
    b2r = gcn_b2[None, :]

    u1a, u1b, dinv = _enc_prep(x, nt2, d02, d12, enc_W1, enc_b1, enc_W2,
                               enc_b2, gcn_W1)
    s1a, s1b = _seg_kernel(u1a, u1b, src2d, dst2d)
    h1, u2a, u2b = _mid(s1a, s1b, u1a, u1b, dinv, gcn_W2, b1r)
    s2a, s2b = _seg_kernel(u2a, u2b, src2d, dst2d)
    return _final(s2a, s2b, u2a, u2b, dinv, h1, b2r)


# speedup vs baseline: 26.6099x; 1.0697x over previous
import functools

import jax
import jax.numpy as jnp
from jax import lax
from jax.experimental import pallas as pl
from jax.experimental.pallas import tpu as pltpu
from jax.experimental.pallas import tpu_sc as plsc

N = 100000
E = 1600000
D_IN = 128
H_ENC = 64
D_ENC = 32
NT = 2

NP = 100096
EPR = 12544
EP = EPR * 128
R = 8
TILE_ROWS_SEG = EPR // 16
NBLK_SEG = TILE_ROWS_SEG // R
TILE_ROWS_DEG = EPR // 32
NBLK_DEG = TILE_ROWS_DEG // R
NPT = NP // 16

_mesh = plsc.VectorSubcoreMesh(core_axis_name="c", subcore_axis_name="s")

f32 = jnp.float32
i32 = jnp.int32


@functools.partial(
    pl.kernel,
    out_type=[jax.ShapeDtypeStruct((NP,), f32),
              jax.ShapeDtypeStruct((NP,), f32)],
    mesh=_mesh,
    compiler_params=pltpu.CompilerParams(use_tc_tiling_on_sc=False),
    scratch_types=[
        pltpu.VMEM((1024,), i32),
        pltpu.VMEM((1024,), f32),
        pltpu.VMEM((1024,), f32),
        pltpu.VMEM_SHARED((NP,), f32),
    ],
)
def _deg_kernel(dst1, o0, o1, didx, ones_v, zb, acc):
    cid = lax.axis_index("c")
    sid = lax.axis_index("s")

    def fill_z(i, _):
        zb[pl.ds(i * 16, 16)] = jnp.zeros((16,), f32)
        ones_v[pl.ds(i * 16, 16)] = jnp.ones((16,), f32)
        return 0
    lax.fori_loop(0, 64, fill_z, 0)

    t0 = sid * NPT
    for k in range(6):
        pltpu.sync_copy(zb, acc.at[pl.ds(t0 + k * 1024, 1024)])
    pltpu.sync_copy(zb.at[pl.ds(0, 112)], acc.at[pl.ds(t0 + 6144, 112)])
    plsc.subcore_barrier()

    w = cid * 16 + sid
    e_base = w * TILE_ROWS_DEG * 128

    def blk(b, _):
        e0 = e_base + b * 1024
        pltpu.sync_copy(dst1.at[pl.ds(e0, 1024)], didx)
        pltpu.sync_copy(ones_v, acc.at[didx], add=True)
        return 0
    lax.fori_loop(0, NBLK_DEG, blk, 0)
    plsc.subcore_barrier()

    def _writeback(out):
        for k in range(6):
            pltpu.sync_copy(acc.at[pl.ds(t0 + k * 1024, 1024)], zb)
            pltpu.sync_copy(zb, out.at[pl.ds(t0 + k * 1024, 1024)])
        pltpu.sync_copy(acc.at[pl.ds(t0 + 6144, 112)], zb.at[pl.ds(0, 112)])
        pltpu.sync_copy(zb.at[pl.ds(0, 112)], out.at[pl.ds(t0 + 6144, 112)])

    @pl.when(cid == 0)
    def _():
        _writeback(o0)

    @pl.when(cid == 1)
    def _():
        _writeback(o1)


@functools.partial(
    pl.kernel,
    out_type=[jax.ShapeDtypeStruct((NP, 16), f32),
              jax.ShapeDtypeStruct((NP, 16), f32)],
    mesh=_mesh,
    compiler_params=pltpu.CompilerParams(use_tc_tiling_on_sc=False),
    scratch_types=[
        pltpu.VMEM((1024,), i32),
        pltpu.VMEM((1024,), i32),
        pltpu.VMEM((1024, 16), f32),
        pltpu.VMEM_SHARED((NP, 16), f32),
        pltpu.SemaphoreType.DMA,
    ],
)
def _seg_kernel(ua, ub, src1, dst1, oa, ob, sidx, didx, rows, acc, sem):
    cid = lax.axis_index("c")
    sid = lax.axis_index("s")
    t0 = sid * NPT

    def run(table, out):
        def fill_z(i, _):
            rows[i, :] = jnp.zeros((16,), f32)
            return 0
        lax.fori_loop(0, R * 128, fill_z, 0)
        for k in range(6):
            pltpu.sync_copy(rows.at[pl.ds(0, 1024)],
                            acc.at[pl.ds(t0 + k * 1024, 1024)])
        pltpu.sync_copy(rows.at[pl.ds(0, 112)],
                        acc.at[pl.ds(t0 + 6144, 112)])
        plsc.subcore_barrier()

        e_base = sid * TILE_ROWS_SEG * 128

        def blk(b, _):
            e0 = e_base + b * 1024
            pltpu.sync_copy(src1.at[pl.ds(e0, 1024)], sidx)
            pltpu.sync_copy(dst1.at[pl.ds(e0, 1024)], didx)
            pltpu.async_copy(table.at[sidx], rows, sem).wait()
            pltpu.sync_copy(rows, acc.at[didx], add=True)
            return 0
        lax.fori_loop(0, NBLK_SEG, blk, 0)
        plsc.subcore_barrier()
        for k in range(6):
            pltpu.sync_copy(acc.at[pl.ds(t0 + k * 1024, 1024)],
                            rows.at[pl.ds(0, 1024)])
            pltpu.sync_copy(rows.at[pl.ds(0, 1024)],
                            out.at[pl.ds(t0 + k * 1024, 1024)])
        pltpu.sync_copy(acc.at[pl.ds(t0 + 6144, 112)],
                        rows.at[pl.ds(0, 112)])
        pltpu.sync_copy(rows.at[pl.ds(0, 112)],
                        out.at[pl.ds(t0 + 6144, 112)])

    @pl.when(cid == 0)
    def _():
        run(ua, oa)

    @pl.when(cid == 1)
    def _():
        run(ub, ob)


BN = 2000
GRID = N // BN


def _enc_prep_body(x_ref, nt_ref, d0_ref, d1_ref, w1_ref, b1_ref, w2_ref,
                   b2_ref, gw1_ref, ua_ref, ub_ref, dinv_ref):
    x = x_ref[...]
    za = jnp.maximum(jnp.dot(x, w1_ref[0], preferred_element_type=f32)
                     + b1_ref[0], 0.0)
    za = jnp.maximum(jnp.dot(za, w2_ref[0], preferred_element_type=f32)
                     + b2_ref[0], 0.0)
    zb = jnp.maximum(jnp.dot(x, w1_ref[1], preferred_element_type=f32)
                     + b1_ref[1], 0.0)
    zb = jnp.maximum(jnp.dot(zb, w2_ref[1], preferred_element_type=f32)
                     + b2_ref[1], 0.0)
    h = jnp.where(nt_ref[...] == 0, za, zb)
    dinv = lax.rsqrt(d0_ref[...] + d1_ref[...] + 1.0)
    u = jnp.dot(h, gw1_ref[...], preferred_element_type=f32) * dinv
    ua_ref[...] = u[:, :16]
    ub_ref[...] = u[:, 16:]
    dinv_ref[...] = dinv


def _enc_prep(x, nt2, d02, d12, enc_W1, enc_b1, enc_W2, enc_b2, gcn_W1):
    return pl.pallas_call(
        _enc_prep_body,
        grid=(GRID,),
        in_specs=[
            pl.BlockSpec((BN, D_IN), lambda i: (i, 0)),
            pl.BlockSpec((BN, 1), lambda i: (i, 0)),
            pl.BlockSpec((BN, 1), lambda i: (i, 0)),
            pl.BlockSpec((BN, 1), lambda i: (i, 0)),
            pl.BlockSpec((NT, D_IN, H_ENC), lambda i: (0, 0, 0)),
            pl.BlockSpec((NT, H_ENC), lambda i: (0, 0)),
            pl.BlockSpec((NT, H_ENC, D_ENC), lambda i: (0, 0, 0)),
            pl.BlockSpec((NT, D_ENC), lambda i: (0, 0)),
            pl.BlockSpec((D_ENC, D_ENC), lambda i: (0, 0)),
        ],
        out_specs=[
            pl.BlockSpec((BN, 16), lambda i: (i, 0)),
            pl.BlockSpec((BN, 16), lambda i: (i, 0)),
            pl.BlockSpec((BN, 1), lambda i: (i, 0)),
        ],
        out_shape=[
            jax.ShapeDtypeStruct((N, 16), f32),
            jax.ShapeDtypeStruct((N, 16), f32),
            jax.ShapeDtypeStruct((N, 1), f32),
        ],
    )(x, nt2, d02, d12, enc_W1, enc_b1, enc_W2, enc_b2, gcn_W1)


def _mid_body(sa_ref, sb_ref, ua_ref, ub_ref, dinv_ref, gw2_ref, b1_ref,
              h1_ref, va_ref, vb_ref):
    dinv = dinv_ref[...]
    t = jnp.concatenate([sa_ref[...] + ua_ref[...],
                         sb_ref[...] + ub_ref[...]], axis=1)
    h1 = jnp.maximum(t * dinv + b1_ref[...], 0.0)
    h1_ref[...] = h1
    v = jnp.dot(h1, gw2_ref[...], preferred_element_type=f32) * dinv
    va_ref[...] = v[:, :16]
    vb_ref[...] = v[:, 16:]


def _mid(s1a, s1b, u1a, u1b, dinv, gcn_W2, b1r):
    return pl.pallas_call(
        _mid_body,
        grid=(GRID,),
        in_specs=[
            pl.BlockSpec((BN, 16), lambda i: (i, 0)),
            pl.BlockSpec((BN, 16), lambda i: (i, 0)),
            pl.BlockSpec((BN, 16), lambda i: (i, 0)),
            pl.BlockSpec((BN, 16), lambda i: (i, 0)),
            pl.BlockSpec((BN, 1), lambda i: (i, 0)),
            pl.BlockSpec((D_ENC, D_ENC), lambda i: (0, 0)),
            pl.BlockSpec((1, D_ENC), lambda i: (0, 0)),
        ],
        out_specs=[
            pl.BlockSpec((BN, D_ENC), lambda i: (i, 0)),
            pl.BlockSpec((BN, 16), lambda i: (i, 0)),
            pl.BlockSpec((BN, 16), lambda i: (i, 0)),
        ],
        out_shape=[
            jax.ShapeDtypeStruct((N, D_ENC), f32),
            jax.ShapeDtypeStruct((N, 16), f32),
            jax.ShapeDtypeStruct((N, 16), f32),
        ],
    )(s1a, s1b, u1a, u1b, dinv, gcn_W2, b1r)


def _final_body(sa_ref, sb_ref, va_ref, vb_ref, dinv_ref, h1_ref, b2_ref,
                out_ref):
    dinv = dinv_ref[...]
    t = jnp.concatenate([sa_ref[...] + va_ref[...],
                         sb_ref[...] + vb_ref[...]], axis=1)
    h2 = jnp.maximum(t * dinv + b2_ref[...], 0.0)
    out_ref[...] = jnp.concatenate([h1_ref[...], h2], axis=1)


def _final(s2a, s2b, u2a, u2b, dinv, h1, b2r):
    return pl.pallas_call(
        _final_body,
        grid=(GRID,),
        in_specs=[
            pl.BlockSpec((BN, 16), lambda i: (i, 0)),
            pl.BlockSpec((BN, 16), lambda i: (i, 0)),
            pl.BlockSpec((BN, 16), lambda i: (i, 0)),
            pl.BlockSpec((BN, 16), lambda i: (i, 0)),
            pl.BlockSpec((BN, 1), lambda i: (i, 0)),
            pl.BlockSpec((BN, D_ENC), lambda i: (i, 0)),
            pl.BlockSpec((1, D_ENC), lambda i: (0, 0)),
        ],
        out_specs=pl.BlockSpec((BN, 2 * D_ENC), lambda i: (i, 0)),
        out_shape=jax.ShapeDtypeStruct((N, 2 * D_ENC), f32),
    )(s2a, s2b, u2a, u2b, dinv, h1, b2r)


def kernel(x, edge_index, node_type, enc_W1, enc_b1, enc_W2, enc_b2,
           gcn_W1, gcn_b1, gcn_W2, gcn_b2):
    pad = EP - E
    pad_src = (jnp.arange(pad, dtype=i32) % 128)
    pad_dst = N + (jnp.arange(pad, dtype=i32) % 16)
    src1 = jnp.concatenate([edge_index[0], pad_src])
    dst1 = jnp.concatenate([edge_index[1], pad_dst])

    d0, d1 = _deg_kernel(dst1)
    d02 = d0[:, None]
    d12 = d1[:, None]

    nt2 = node_type[:, None]
    b1r = gcn_b1[None, :]
    b2r = gcn_b2[None, :]

    u1a, u1b, dinv = _enc_prep(x, nt2, d02, d12, enc_W1, enc_b1, enc_W2,
                               enc_b2, gcn_W1)
    s1a, s1b = _seg_kernel(u1a, u1b, src1, dst1)
    h1, u2a, u2b = _mid(s1a, s1b, u1a, u1b, dinv, gcn_W2, b1r)
    s2a, s2b = _seg_kernel(u2a, u2b, src1, dst1)
    return _final(s2a, s2b, u2a, u2b, dinv, h1, b2r)


# speedup vs baseline: 34.8095x; 1.3081x over previous
import functools

import jax
import jax.numpy as jnp
from jax import lax
from jax.experimental import pallas as pl
from jax.experimental.pallas import tpu as pltpu
from jax.experimental.pallas import tpu_sc as plsc

N = 100000
E = 1600000
D_IN = 128
H_ENC = 64
D_ENC = 32
NT = 2

NP_D = 100096
NPT_D = NP_D // 16
BLK = 512

ET_SEG = E // 16
NBF_SEG = ET_SEG // BLK
TL_SEG = ET_SEG - NBF_SEG * BLK

ET_DEG = E // 32
NBF_DEG = ET_DEG // BLK
TL_DEG = ET_DEG - NBF_DEG * BLK
NQ_DEG = NBF_DEG // 4

NZC = NPT_D // BLK
NZT = NPT_D - NZC * BLK

BN = 2000
GRID = N // BN

_mesh = plsc.VectorSubcoreMesh(core_axis_name="c", subcore_axis_name="s")

f32 = jnp.float32
i32 = jnp.int32


@functools.partial(
    pl.kernel,
    out_type=[jax.ShapeDtypeStruct((NP_D,), f32),
              jax.ShapeDtypeStruct((NP_D,), f32)],
    mesh=_mesh,
    compiler_params=pltpu.CompilerParams(use_tc_tiling_on_sc=False),
    scratch_types=[
        [pltpu.VMEM((BLK,), i32) for _ in range(4)],
        pltpu.VMEM((TL_DEG,), i32),
        pltpu.VMEM((BLK,), f32),
        pltpu.VMEM((BLK,), f32),
        pltpu.VMEM_SHARED((NP_D,), f32),
        [pltpu.SemaphoreType.DMA for _ in range(4)],
        [pltpu.SemaphoreType.DMA for _ in range(4)],
    ],
)
def _deg_kernel(dst1, o0, o1, didx, didx_t, ones_v, zb, acc, smi, sms):
    cid = lax.axis_index("c")
    sid = lax.axis_index("s")

    def fill_z(i, _):
        zb[pl.ds(i * 16, 16)] = jnp.zeros((16,), f32)
        ones_v[pl.ds(i * 16, 16)] = jnp.ones((16,), f32)
        return 0
    lax.fori_loop(0, BLK // 16, fill_z, 0)

    t0 = sid * NPT_D
    for k in range(NZC):
        pltpu.sync_copy(zb, acc.at[pl.ds(t0 + k * BLK, BLK)])
    pltpu.sync_copy(zb.at[pl.ds(0, NZT)],
                    acc.at[pl.ds(t0 + NZC * BLK, NZT)])
    plsc.subcore_barrier()

    w = cid * 16 + sid
    e_base = w * ET_DEG

    for x in range(4):
        pltpu.async_copy(dst1.at[pl.ds(e_base + x * BLK, BLK)],
                         didx[x], smi[x])

    def quad(k, _):
        for x in range(4):
            pltpu.make_async_copy(dst1.at[pl.ds(e_base, BLK)],
                                  didx[x], smi[x]).wait()
            pltpu.async_copy(ones_v, acc.at[didx[x]], sms[x], add=True)
        for x in range(4):
            pltpu.make_async_copy(ones_v, acc.at[didx[x]], sms[x]).wait()

            @pl.when(k + 1 < NQ_DEG)
            def _():
                e0 = e_base + (4 * (k + 1) + x) * BLK
                pltpu.async_copy(dst1.at[pl.ds(e0, BLK)], didx[x], smi[x])
        return 0
    lax.fori_loop(0, NQ_DEG, quad, 0)
    pltpu.sync_copy(dst1.at[pl.ds(e_base + (NBF_DEG - 1) * BLK, BLK)],
                    didx[0])
    pltpu.sync_copy(ones_v, acc.at[didx[0]], add=True)
    pltpu.sync_copy(dst1.at[pl.ds(e_base + NBF_DEG * BLK, TL_DEG)], didx_t)
    pltpu.sync_copy(ones_v.at[pl.ds(0, TL_DEG)], acc.at[didx_t], add=True)
    plsc.subcore_barrier()

    def _writeback(out):
        for k in range(NZC):
            pltpu.sync_copy(acc.at[pl.ds(t0 + k * BLK, BLK)], zb)
            pltpu.sync_copy(zb, out.at[pl.ds(t0 + k * BLK, BLK)])
        pltpu.sync_copy(acc.at[pl.ds(t0 + NZC * BLK, NZT)],
                        zb.at[pl.ds(0, NZT)])
        pltpu.sync_copy(zb.at[pl.ds(0, NZT)],
                        out.at[pl.ds(t0 + NZC * BLK, NZT)])

    @pl.when(cid == 0)
    def _():
        _writeback(o0)

    @pl.when(cid == 1)
    def _():
        _writeback(o1)


@functools.partial(
    pl.kernel,
    out_type=[jax.ShapeDtypeStruct((NP_D, 16), f32),
              jax.ShapeDtypeStruct((NP_D, 16), f32)],
    mesh=_mesh,
    compiler_params=pltpu.CompilerParams(use_tc_tiling_on_sc=False),
    scratch_types=[
        [pltpu.VMEM((BLK,), i32) for _ in range(2)],
        [pltpu.VMEM((BLK,), i32) for _ in range(2)],
        [pltpu.VMEM((BLK, 16), f32) for _ in range(2)],
        pltpu.VMEM((TL_SEG,), i32),
        pltpu.VMEM((TL_SEG,), i32),
        pltpu.VMEM((TL_SEG, 16), f32),
        pltpu.VMEM_SHARED((NP_D, 16), f32),
        [pltpu.SemaphoreType.DMA for _ in range(2)],
        [pltpu.SemaphoreType.DMA for _ in range(2)],
        [pltpu.SemaphoreType.DMA for _ in range(2)],
        [pltpu.SemaphoreType.DMA for _ in range(2)],
    ],
)
def _seg_kernel(ua, ub, src1, dst1, oa, ob, sidx, didx, rows,
                sidx_t, didx_t, rows_t, acc, smi_s, smi_d, smg, sms):
    cid = lax.axis_index("c")
    sid = lax.axis_index("s")
    t0 = sid * NPT_D

    def run(table, out):
        def init_chunk(k, sz):
            pltpu.sync_copy(table.at[pl.ds(t0 + k * BLK, sz)],
                            rows[0].at[pl.ds(0, sz)])
            pltpu.sync_copy(rows[0].at[pl.ds(0, sz)],
                            acc.at[pl.ds(t0 + k * BLK, sz)])

        for k in range(NZC):
            init_chunk(k, BLK)

        @pl.when(sid < 15)
        def _():
            init_chunk(NZC, NZT)

        @pl.when(sid == 15)
        def _():
            base = 15 * NPT_D + NZC * BLK
            pltpu.sync_copy(table.at[pl.ds(base, 16)],
                            rows[0].at[pl.ds(0, 16)])

            def zfill(i, _):
                rows[0][16 + i, :] = jnp.zeros((16,), f32)
                return 0
            lax.fori_loop(0, NZT - 16, zfill, 0)
            pltpu.sync_copy(rows[0].at[pl.ds(0, NZT)],
                            acc.at[pl.ds(base, NZT)])
        plsc.subcore_barrier()

        e_base = sid * ET_SEG

        pltpu.async_copy(src1.at[pl.ds(e_base, BLK)], sidx[0], smi_s[0])
        pltpu.async_copy(src1.at[pl.ds(e_base + BLK, BLK)],
                         sidx[1], smi_s[1])

        def sub(b, x):
            e0 = e_base + b * BLK

            @pl.when(b >= 2)
            def _():
                pltpu.make_async_copy(rows[x], acc.at[didx[x]],
                                      sms[x]).wait()
            pltpu.async_copy(dst1.at[pl.ds(e0, BLK)], didx[x], smi_d[x])
            pltpu.make_async_copy(src1.at[pl.ds(e0, BLK)],
                                  sidx[x], smi_s[x]).wait()
            g = pltpu.async_copy(table.at[sidx[x]], rows[x], smg[x])
            g.wait()

            @pl.when(b + 2 < NBF_SEG)
            def _():
                pltpu.async_copy(src1.at[pl.ds(e0 + 2 * BLK, BLK)],
                                 sidx[x], smi_s[x])
            pltpu.make_async_copy(dst1.at[pl.ds(e0, BLK)],
                                  didx[x], smi_d[x]).wait()
            pltpu.async_copy(rows[x], acc.at[didx[x]], sms[x], add=True)

        def pair(k, _):
            sub(2 * k, 0)
            sub(2 * k + 1, 1)
            return 0
        lax.fori_loop(0, NBF_SEG // 2, pair, 0)

        b194 = e_base + (NBF_SEG - 1) * BLK
        pltpu.make_async_copy(rows[0], acc.at[didx[0]], sms[0]).wait()
        pltpu.async_copy(dst1.at[pl.ds(b194, BLK)], didx[0], smi_d[0])
        pltpu.make_async_copy(src1.at[pl.ds(b194, BLK)],
                              sidx[0], smi_s[0]).wait()
        pltpu.async_copy(table.at[sidx[0]], rows[0], smg[0]).wait()
        pltpu.make_async_copy(dst1.at[pl.ds(b194, BLK)],
                              didx[0], smi_d[0]).wait()
        pltpu.async_copy(rows[0], acc.at[didx[0]], sms[0], add=True)

        et = e_base + NBF_SEG * BLK
        pltpu.sync_copy(src1.at[pl.ds(et, TL_SEG)], sidx_t)
        pltpu.sync_copy(dst1.at[pl.ds(et, TL_SEG)], didx_t)
        pltpu.async_copy(table.at[sidx_t], rows_t, smg[1]).wait()
        pltpu.sync_copy(rows_t, acc.at[didx_t], add=True)

        pltpu.make_async_copy(rows[1], acc.at[didx[1]], sms[1]).wait()
        pltpu.make_async_copy(rows[0], acc.at[didx[0]], sms[0]).wait()
        plsc.subcore_barrier()

        def wb_chunk(k, sz):
            pltpu.sync_copy(acc.at[pl.ds(t0 + k * BLK, sz)],
                            rows[0].at[pl.ds(0, sz)])
            pltpu.sync_copy(rows[0].at[pl.ds(0, sz)],
                            out.at[pl.ds(t0 + k * BLK, sz)])

        for k in range(NZC):
            wb_chunk(k, BLK)
        wb_chunk(NZC, NZT)

    @pl.when(cid == 0)
    def _():
        run(ua, oa)

    @pl.when(cid == 1)
    def _():
        run(ub, ob)


def _col(ref):
    return jnp.sum(ref[...], axis=(0, 1))[:, None]


_SCAL = pl.BlockSpec((1, 1, BN), lambda i: (i, 0, 0))
_ROWS16 = pl.BlockSpec((BN, 16), lambda i: (i, 0))
_ROWS32 = pl.BlockSpec((BN, D_ENC), lambda i: (i, 0))
_W22 = pl.BlockSpec((D_ENC, D_ENC), lambda i: (0, 0))
_B32 = pl.BlockSpec((1, D_ENC), lambda i: (0, 0))


def _enc_prep_body(x_ref, nt_ref, d0_ref, d1_ref, w1_ref, b1_ref, w2_ref,
                   b2_ref, gw1_ref, ua_ref, ub_ref):
    x = x_ref[...]
    za = jnp.maximum(jnp.dot(x, w1_ref[0], preferred_element_type=f32)
                     + b1_ref[0], 0.0)
    za = jnp.maximum(jnp.dot(za, w2_ref[0], preferred_element_type=f32)
                     + b2_ref[0], 0.0)
    zb = jnp.maximum(jnp.dot(x, w1_ref[1], preferred_element_type=f32)
                     + b1_ref[1], 0.0)
    zb = jnp.maximum(jnp.dot(zb, w2_ref[1], preferred_element_type=f32)
                     + b2_ref[1], 0.0)
    h = jnp.where(_col(nt_ref) == 0, za, zb)
    dinv = lax.rsqrt(_col(d0_ref) + _col(d1_ref) + 1.0)
    u = jnp.dot(h, gw1_ref[...], preferred_element_type=f32) * dinv
    ua_ref[...] = u[:, :16]
    ub_ref[...] = u[:, 16:]


def _enc_prep(x, nt50, d050, d150, enc_W1, enc_b1, enc_W2, enc_b2, gcn_W1):
    return pl.pallas_call(
        _enc_prep_body,
        grid=(GRID,),
        in_specs=[
            pl.BlockSpec((BN, D_IN), lambda i: (i, 0)),
            _SCAL, _SCAL, _SCAL,
            pl.BlockSpec((NT, D_IN, H_ENC), lambda i: (0, 0, 0)),
            pl.BlockSpec((NT, H_ENC), lambda i: (0, 0)),
            pl.BlockSpec((NT, H_ENC, D_ENC), lambda i: (0, 0, 0)),
            pl.BlockSpec((NT, D_ENC), lambda i: (0, 0)),
            _W22,
        ],
        out_specs=[_ROWS16, _ROWS16],
        out_shape=[
            jax.ShapeDtypeStruct((N, 16), f32),
            jax.ShapeDtypeStruct((N, 16), f32),
        ],
    )(x, nt50, d050, d150, enc_W1, enc_b1, enc_W2, enc_b2, gcn_W1)


def _mid_body(sa_ref, sb_ref, d0_ref, d1_ref, gw2_ref, b1_ref,
              h1_ref, va_ref, vb_ref):
    dinv = lax.rsqrt(_col(d0_ref) + _col(d1_ref) + 1.0)
    t = jnp.concatenate([sa_ref[...], sb_ref[...]], axis=1)
    h1 = jnp.maximum(t * dinv + b1_ref[...], 0.0)
    h1_ref[...] = h1
    v = jnp.dot(h1, gw2_ref[...], preferred_element_type=f32) * dinv
    va_ref[...] = v[:, :16]
    vb_ref[...] = v[:, 16:]


def _mid(s1a, s1b, d050, d150, gcn_W2, b1r):
    return pl.pallas_call(
        _mid_body,
        grid=(GRID,),
        in_specs=[_ROWS16, _ROWS16, _SCAL, _SCAL, _W22, _B32],
        out_specs=[_ROWS32, _ROWS16, _ROWS16],
        out_shape=[
            jax.ShapeDtypeStruct((N, D_ENC), f32),
            jax.ShapeDtypeStruct((N, 16), f32),
            jax.ShapeDtypeStruct((N, 16), f32),
        ],
    )(s1a, s1b, d050, d150, gcn_W2, b1r)


def _final_body(sa_ref, sb_ref, d0_ref, d1_ref, h1_ref, b2_ref, out_ref):
    dinv = lax.rsqrt(_col(d0_ref) + _col(d1_ref) + 1.0)
    t = jnp.concatenate([sa_ref[...], sb_ref[...]], axis=1)
    h2 = jnp.maximum(t * dinv + b2_ref[...], 0.0)
    out_ref[...] = jnp.concatenate([h1_ref[...], h2], axis=1)


def _final(s2a, s2b, d050, d150, h1, b2r):
    return pl.pallas_call(
        _final_body,
        grid=(GRID,),
        in_specs=[_ROWS16, _ROWS16, _SCAL, _SCAL, _ROWS32, _B32],
        out_specs=pl.BlockSpec((BN, 2 * D_ENC), lambda i: (i, 0)),
        out_shape=jax.ShapeDtypeStruct((N, 2 * D_ENC), f32),
    )(s2a, s2b, d050, d150, h1, b2r)


def kernel(x, edge_index, node_type, enc_W1, enc_b1, enc_W2, enc_b2,
           gcn_W1, gcn_b1, gcn_W2, gcn_b2):
    src1 = edge_index[0]
    dst1 = edge_index[1]

    d0, d1 = _deg_kernel(dst1)
    d0b = lax.optimization_barrier(
        lax.broadcast_in_dim(d0, (1, NP_D), (1,)))
    d1b = lax.optimization_barrier(
        lax.broadcast_in_dim(d1, (1, NP_D), (1,)))
    d050 = d0b[:, :N].reshape(GRID, 1, BN)
    d150 = d1b[:, :N].reshape(GRID, 1, BN)

    nt50 = node_type.reshape(GRID, 1, BN)
    b1r = gcn_b1[None, :]
    b2r = gcn_b2[None, :]

    u1a, u1b = _enc_prep(x, nt50, d050, d150, enc_W1, enc_b1,
                         enc_W2, enc_b2, gcn_W1)
    s1a, s1b = _seg_kernel(u1a, u1b, src1, dst1)
    h1, u2a, u2b = _mid(s1a, s1b, d050, d150, gcn_W2, b1r)
    s2a, s2b = _seg_kernel(u2a, u2b, src1, dst1)
    return _final(s2a, s2b, d050, d150, h1, b2r)


# speedup vs baseline: 38.2543x; 1.0990x over previous
import functools

import jax
import jax.numpy as jnp
from jax import lax
from jax.experimental import pallas as pl
from jax.experimental.pallas import tpu as pltpu
from jax.experimental.pallas import tpu_sc as plsc

N = 100000
E = 1600000
D_IN = 128
H_ENC = 64
D_ENC = 32
NT = 2

NP_D = 100096
NPT_D = NP_D // 16
BLK = 512

ET_SEG = E // 16
NBF_SEG = ET_SEG // BLK
TL_SEG = ET_SEG - NBF_SEG * BLK

ET_DEG = E // 32
NBF_DEG = ET_DEG // BLK
TL_DEG = ET_DEG - NBF_DEG * BLK
NQ_DEG = NBF_DEG // 4

NZC = NPT_D // BLK
NZT = NPT_D - NZC * BLK

BN = 2000
GRID = N // BN

_mesh = plsc.VectorSubcoreMesh(core_axis_name="c", subcore_axis_name="s")

f32 = jnp.float32
i32 = jnp.int32


@functools.partial(
    pl.kernel,
    out_type=[jax.ShapeDtypeStruct((NP_D,), f32),
              jax.ShapeDtypeStruct((NP_D,), f32)],
    mesh=_mesh,
    compiler_params=pltpu.CompilerParams(use_tc_tiling_on_sc=False),
    scratch_types=[
        [pltpu.VMEM((BLK,), i32) for _ in range(4)],
        pltpu.VMEM((TL_DEG,), i32),
        pltpu.VMEM((BLK,), f32),
        pltpu.VMEM((BLK,), f32),
        pltpu.VMEM_SHARED((NP_D,), f32),
        [pltpu.SemaphoreType.DMA for _ in range(4)],
        [pltpu.SemaphoreType.DMA for _ in range(4)],
    ],
)
def _deg_kernel(dst1, o0, o1, didx, didx_t, ones_v, zb, acc, smi, sms):
    cid = lax.axis_index("c")
    sid = lax.axis_index("s")

    def fill_z(i, _):
        zb[pl.ds(i * 16, 16)] = jnp.zeros((16,), f32)
        ones_v[pl.ds(i * 16, 16)] = jnp.ones((16,), f32)
        return 0
    lax.fori_loop(0, BLK // 16, fill_z, 0)

    t0 = sid * NPT_D
    for k in range(NZC):
        pltpu.sync_copy(zb, acc.at[pl.ds(t0 + k * BLK, BLK)])
    pltpu.sync_copy(zb.at[pl.ds(0, NZT)],
                    acc.at[pl.ds(t0 + NZC * BLK, NZT)])
    plsc.subcore_barrier()

    w = cid * 16 + sid
    e_base = w * ET_DEG

    for x in range(4):
        pltpu.async_copy(dst1.at[pl.ds(e_base + x * BLK, BLK)],
                         didx[x], smi[x])

    def quad(k, _):
        for x in range(4):
            pltpu.make_async_copy(dst1.at[pl.ds(e_base, BLK)],
                                  didx[x], smi[x]).wait()
            pltpu.async_copy(ones_v, acc.at[didx[x]], sms[x], add=True)
        for x in range(4):
            pltpu.make_async_copy(ones_v, acc.at[didx[x]], sms[x]).wait()

            @pl.when(k + 1 < NQ_DEG)
            def _():
                e0 = e_base + (4 * (k + 1) + x) * BLK
                pltpu.async_copy(dst1.at[pl.ds(e0, BLK)], didx[x], smi[x])
        return 0
    lax.fori_loop(0, NQ_DEG, quad, 0)
    pltpu.sync_copy(dst1.at[pl.ds(e_base + (NBF_DEG - 1) * BLK, BLK)],
                    didx[0])
    pltpu.sync_copy(ones_v, acc.at[didx[0]], add=True)
    pltpu.sync_copy(dst1.at[pl.ds(e_base + NBF_DEG * BLK, TL_DEG)], didx_t)
    pltpu.sync_copy(ones_v.at[pl.ds(0, TL_DEG)], acc.at[didx_t], add=True)
    plsc.subcore_barrier()

    def _writeback(out):
        for k in range(NZC):
            pltpu.sync_copy(acc.at[pl.ds(t0 + k * BLK, BLK)], zb)
            pltpu.sync_copy(zb, out.at[pl.ds(t0 + k * BLK, BLK)])
        pltpu.sync_copy(acc.at[pl.ds(t0 + NZC * BLK, NZT)],
                        zb.at[pl.ds(0, NZT)])
        pltpu.sync_copy(zb.at[pl.ds(0, NZT)],
                        out.at[pl.ds(t0 + NZC * BLK, NZT)])

    @pl.when(cid == 0)
    def _():
        _writeback(o0)

    @pl.when(cid == 1)
    def _():
        _writeback(o1)


@functools.partial(
    pl.kernel,
    out_type=[jax.ShapeDtypeStruct((NP_D, 16), f32),
              jax.ShapeDtypeStruct((NP_D, 16), f32)],
    mesh=_mesh,
    compiler_params=pltpu.CompilerParams(use_tc_tiling_on_sc=False),
    scratch_types=[
        [pltpu.VMEM((BLK,), i32) for _ in range(2)],
        [pltpu.VMEM((BLK,), i32) for _ in range(2)],
        [pltpu.VMEM((BLK, 16), f32) for _ in range(2)],
        pltpu.VMEM((TL_SEG,), i32),
        pltpu.VMEM((TL_SEG,), i32),
        pltpu.VMEM((TL_SEG, 16), f32),
        pltpu.VMEM_SHARED((NP_D, 16), f32),
        [pltpu.SemaphoreType.DMA for _ in range(2)],
        [pltpu.SemaphoreType.DMA for _ in range(2)],
        [pltpu.SemaphoreType.DMA for _ in range(2)],
        [pltpu.SemaphoreType.DMA for _ in range(2)],
    ],
)
def _seg_kernel(ua, ub, src1, dst1, oa, ob, sidx, didx, rows,
                sidx_t, didx_t, rows_t, acc, smi_s, smi_d, smg, sms):
    cid = lax.axis_index("c")
    sid = lax.axis_index("s")
    t0 = sid * NPT_D

    def run(table, out):
        def init_chunk(k, sz):
            pltpu.sync_copy(table.at[pl.ds(t0 + k * BLK, sz)],
                            rows[0].at[pl.ds(0, sz)])
            pltpu.sync_copy(rows[0].at[pl.ds(0, sz)],
                            acc.at[pl.ds(t0 + k * BLK, sz)])

        for k in range(NZC):
            init_chunk(k, BLK)

        @pl.when(sid < 15)
        def _():
            init_chunk(NZC, NZT)

        @pl.when(sid == 15)
        def _():
            base = 15 * NPT_D + NZC * BLK
            pltpu.sync_copy(table.at[pl.ds(base, 16)],
                            rows[0].at[pl.ds(0, 16)])

            def zfill(i, _):
                rows[0][16 + i, :] = jnp.zeros((16,), f32)
                return 0
            lax.fori_loop(0, NZT - 16, zfill, 0)
            pltpu.sync_copy(rows[0].at[pl.ds(0, NZT)],
                            acc.at[pl.ds(base, NZT)])
        plsc.subcore_barrier()

        e_base = sid * ET_SEG

        pltpu.async_copy(src1.at[pl.ds(e_base, BLK)], sidx[0], smi_s[0])
        pltpu.async_copy(src1.at[pl.ds(e_base + BLK, BLK)],
                         sidx[1], smi_s[1])

        def sub_fire(b, x):
            e0 = e_base + b * BLK

            @pl.when(b >= 2)
            def _():
                pltpu.make_async_copy(rows[x], acc.at[didx[x]],
                                      sms[x]).wait()
            pltpu.async_copy(dst1.at[pl.ds(e0, BLK)], didx[x], smi_d[x])
            pltpu.make_async_copy(src1.at[pl.ds(e0, BLK)],
                                  sidx[x], smi_s[x]).wait()
            pltpu.async_copy(table.at[sidx[x]], rows[x], smg[x])

        def sub_drain(b, x):
            e0 = e_base + b * BLK
            pltpu.make_async_copy(table.at[sidx[x]], rows[x],
                                  smg[x]).wait()

            @pl.when(b + 2 < NBF_SEG)
            def _():
                pltpu.async_copy(src1.at[pl.ds(e0 + 2 * BLK, BLK)],
                                 sidx[x], smi_s[x])
            pltpu.make_async_copy(dst1.at[pl.ds(e0, BLK)],
                                  didx[x], smi_d[x]).wait()
            pltpu.async_copy(rows[x], acc.at[didx[x]], sms[x], add=True)

        def pair(k, _):
            sub_fire(2 * k, 0)
            sub_fire(2 * k + 1, 1)
            sub_drain(2 * k, 0)
            sub_drain(2 * k + 1, 1)
            return 0
        lax.fori_loop(0, NBF_SEG // 2, pair, 0)

        b194 = e_base + (NBF_SEG - 1) * BLK
        pltpu.make_async_copy(rows[0], acc.at[didx[0]], sms[0]).wait()
        pltpu.async_copy(dst1.at[pl.ds(b194, BLK)], didx[0], smi_d[0])
        pltpu.make_async_copy(src1.at[pl.ds(b194, BLK)],
                              sidx[0], smi_s[0]).wait()
        pltpu.async_copy(table.at[sidx[0]], rows[0], smg[0]).wait()
        pltpu.make_async_copy(dst1.at[pl.ds(b194, BLK)],
                              didx[0], smi_d[0]).wait()
        pltpu.async_copy(rows[0], acc.at[didx[0]], sms[0], add=True)

        et = e_base + NBF_SEG * BLK
        pltpu.sync_copy(src1.at[pl.ds(et, TL_SEG)], sidx_t)
        pltpu.sync_copy(dst1.at[pl.ds(et, TL_SEG)], didx_t)
        pltpu.async_copy(table.at[sidx_t], rows_t, smg[1]).wait()
        pltpu.sync_copy(rows_t, acc.at[didx_t], add=True)

        pltpu.make_async_copy(rows[1], acc.at[didx[1]], sms[1]).wait()
        pltpu.make_async_copy(rows[0], acc.at[didx[0]], sms[0]).wait()
        plsc.subcore_barrier()

        def wb_chunk(k, sz):
            pltpu.sync_copy(acc.at[pl.ds(t0 + k * BLK, sz)],
                            rows[0].at[pl.ds(0, sz)])
            pltpu.sync_copy(rows[0].at[pl.ds(0, sz)],
                            out.at[pl.ds(t0 + k * BLK, sz)])

        for k in range(NZC):
            wb_chunk(k, BLK)
        wb_chunk(NZC, NZT)

    @pl.when(cid == 0)
    def _():
        run(ua, oa)

    @pl.when(cid == 1)
    def _():
        run(ub, ob)


def _col(ref):
    return jnp.sum(ref[...], axis=(0, 1))[:, None]


_SCAL = pl.BlockSpec((1, 1, BN), lambda i: (i, 0, 0))
_ROWS16 = pl.BlockSpec((BN, 16), lambda i: (i, 0))
_ROWS32 = pl.BlockSpec((BN, D_ENC), lambda i: (i, 0))
_W22 = pl.BlockSpec((D_ENC, D_ENC), lambda i: (0, 0))
_B32 = pl.BlockSpec((1, D_ENC), lambda i: (0, 0))


def _enc_prep_body(x_ref, nt_ref, d0_ref, d1_ref, w1_ref, b1_ref, w2_ref,
                   b2_ref, gw1_ref, ua_ref, ub_ref):
    x = x_ref[...]
    za = jnp.maximum(jnp.dot(x, w1_ref[0], preferred_element_type=f32)
                     + b1_ref[0], 0.0)
    za = jnp.maximum(jnp.dot(za, w2_ref[0], preferred_element_type=f32)
                     + b2_ref[0], 0.0)
    zb = jnp.maximum(jnp.dot(x, w1_ref[1], preferred_element_type=f32)
                     + b1_ref[1], 0.0)
    zb = jnp.maximum(jnp.dot(zb, w2_ref[1], preferred_element_type=f32)
                     + b2_ref[1], 0.0)
    h = jnp.where(_col(nt_ref) == 0, za, zb)
    dinv = lax.rsqrt(_col(d0_ref) + _col(d1_ref) + 1.0)
    u = jnp.dot(h, gw1_ref[...], preferred_element_type=f32) * dinv
    ua_ref[...] = u[:, :16]
    ub_ref[...] = u[:, 16:]


def _enc_prep(x, nt50, d050, d150, enc_W1, enc_b1, enc_W2, enc_b2, gcn_W1):
    return pl.pallas_call(
        _enc_prep_body,
        grid=(GRID,),
        in_specs=[
            pl.BlockSpec((BN, D_IN), lambda i: (i, 0)),
            _SCAL, _SCAL, _SCAL,
            pl.BlockSpec((NT, D_IN, H_ENC), lambda i: (0, 0, 0)),
            pl.BlockSpec((NT, H_ENC), lambda i: (0, 0)),
            pl.BlockSpec((NT, H_ENC, D_ENC), lambda i: (0, 0, 0)),
            pl.BlockSpec((NT, D_ENC), lambda i: (0, 0)),
            _W22,
        ],
        out_specs=[_ROWS16, _ROWS16],
        out_shape=[
            jax.ShapeDtypeStruct((N, 16), f32),
            jax.ShapeDtypeStruct((N, 16), f32),
        ],
    )(x, nt50, d050, d150, enc_W1, enc_b1, enc_W2, enc_b2, gcn_W1)


def _mid_body(sa_ref, sb_ref, d0_ref, d1_ref, gw2_ref, b1_ref,
              h1_ref, va_ref, vb_ref):
    dinv = lax.rsqrt(_col(d0_ref) + _col(d1_ref) + 1.0)
    t = jnp.concatenate([sa_ref[...], sb_ref[...]], axis=1)
    h1 = jnp.maximum(t * dinv + b1_ref[...], 0.0)
    h1_ref[...] = h1
    v = jnp.dot(h1, gw2_ref[...], preferred_element_type=f32) * dinv
    va_ref[...] = v[:, :16]
    vb_ref[...] = v[:, 16:]


def _mid(s1a, s1b, d050, d150, gcn_W2, b1r):
    return pl.pallas_call(
        _mid_body,
        grid=(GRID,),
        in_specs=[_ROWS16, _ROWS16, _SCAL, _SCAL, _W22, _B32],
        out_specs=[_ROWS32, _ROWS16, _ROWS16],
        out_shape=[
            jax.ShapeDtypeStruct((N, D_ENC), f32),
            jax.ShapeDtypeStruct((N, 16), f32),
            jax.ShapeDtypeStruct((N, 16), f32),
        ],
    )(s1a, s1b, d050, d150, gcn_W2, b1r)


def _final_body(sa_ref, sb_ref, d0_ref, d1_ref, h1_ref, b2_ref, out_ref):
    dinv = lax.rsqrt(_col(d0_ref) + _col(d1_ref) + 1.0)
    t = jnp.concatenate([sa_ref[...], sb_ref[...]], axis=1)
    h2 = jnp.maximum(t * dinv + b2_ref[...], 0.0)
    out_ref[...] = jnp.concatenate([h1_ref[...], h2], axis=1)


def _final(s2a, s2b, d050, d150, h1, b2r):
    return pl.pallas_call(
        _final_body,
        grid=(GRID,),
        in_specs=[_ROWS16, _ROWS16, _SCAL, _SCAL, _ROWS32, _B32],
        out_specs=pl.BlockSpec((BN, 2 * D_ENC), lambda i: (i, 0)),
        out_shape=jax.ShapeDtypeStruct((N, 2 * D_ENC), f32),
    )(s2a, s2b, d050, d150, h1, b2r)


def kernel(x, edge_index, node_type, enc_W1, enc_b1, enc_W2, enc_b2,
           gcn_W1, gcn_b1, gcn_W2, gcn_b2):
    src1 = edge_index[0]
    dst1 = edge_index[1]

    d0, d1 = _deg_kernel(dst1)
    d0b = lax.optimization_barrier(
        lax.broadcast_in_dim(d0, (1, NP_D), (1,)))
    d1b = lax.optimization_barrier(
        lax.broadcast_in_dim(d1, (1, NP_D), (1,)))
    d050 = d0b[:, :N].reshape(GRID, 1, BN)
    d150 = d1b[:, :N].reshape(GRID, 1, BN)

    nt50 = node_type.reshape(GRID, 1, BN)
    b1r = gcn_b1[None, :]
    b2r = gcn_b2[None, :]

    u1a, u1b = _enc_prep(x, nt50, d050, d150, enc_W1, enc_b1,
                         enc_W2, enc_b2, gcn_W1)
    s1a, s1b = _seg_kernel(u1a, u1b, src1, dst1)
    h1, u2a, u2b = _mid(s1a, s1b, d050, d150, gcn_W2, b1r)
    s2a, s2b = _seg_kernel(u2a, u2b, src1, dst1)
    return _final(s2a, s2b, d050, d150, h1, b2r)


# speedup vs baseline: 41.9069x; 1.0955x over previous
import functools

import jax
import jax.numpy as jnp
from jax import lax
from jax.experimental import pallas as pl
from jax.experimental.pallas import tpu as pltpu
from jax.experimental.pallas import tpu_sc as plsc

N = 100000
E = 1600000
D_IN = 128
H_ENC = 64
D_ENC = 32
NT = 2

NP_D = 100096
NPT_D = NP_D // 16
BLK = 512

ET_SEG = E // 16
NBF_SEG = ET_SEG // BLK
TL_SEG = ET_SEG - NBF_SEG * BLK

ET_DEG = E // 32
NBF_DEG = ET_DEG // BLK
TL_DEG = ET_DEG - NBF_DEG * BLK
NQ_DEG = NBF_DEG // 4

NZC = NPT_D // BLK
NZT = NPT_D - NZC * BLK

BN = 2000
GRID = N // BN

_mesh = plsc.VectorSubcoreMesh(core_axis_name="c", subcore_axis_name="s")

f32 = jnp.float32
i32 = jnp.int32


@functools.partial(
    pl.kernel,
    out_type=[jax.ShapeDtypeStruct((NP_D,), f32),
              jax.ShapeDtypeStruct((NP_D,), f32)],
    mesh=_mesh,
    compiler_params=pltpu.CompilerParams(use_tc_tiling_on_sc=False),
    scratch_types=[
        [pltpu.VMEM((BLK,), i32) for _ in range(4)],
        pltpu.VMEM((TL_DEG,), i32),
        pltpu.VMEM((BLK,), f32),
        pltpu.VMEM((BLK,), f32),
        pltpu.VMEM_SHARED((NP_D,), f32),
        [pltpu.SemaphoreType.DMA for _ in range(4)],
        [pltpu.SemaphoreType.DMA for _ in range(4)],
    ],
)
def _deg_kernel(dst1, o0, o1, didx, didx_t, ones_v, zb, acc, smi, sms):
    cid = lax.axis_index("c")
    sid = lax.axis_index("s")

    def fill_z(i, _):
        zb[pl.ds(i * 16, 16)] = jnp.zeros((16,), f32)
        ones_v[pl.ds(i * 16, 16)] = jnp.ones((16,), f32)
        return 0
    lax.fori_loop(0, BLK // 16, fill_z, 0)

    t0 = sid * NPT_D
    for k in range(NZC):
        pltpu.sync_copy(zb, acc.at[pl.ds(t0 + k * BLK, BLK)])
    pltpu.sync_copy(zb.at[pl.ds(0, NZT)],
                    acc.at[pl.ds(t0 + NZC * BLK, NZT)])
    plsc.subcore_barrier()

    w = cid * 16 + sid
    e_base = w * ET_DEG

    for x in range(4):
        pltpu.async_copy(dst1.at[pl.ds(e_base + x * BLK, BLK)],
                         didx[x], smi[x])

    def quad(k, _):
        for x in range(4):
            pltpu.make_async_copy(dst1.at[pl.ds(e_base, BLK)],
                                  didx[x], smi[x]).wait()
            pltpu.async_copy(ones_v, acc.at[didx[x]], sms[x], add=True)
        for x in range(4):
            pltpu.make_async_copy(ones_v, acc.at[didx[x]], sms[x]).wait()

            @pl.when(k + 1 < NQ_DEG)
            def _():
                e0 = e_base + (4 * (k + 1) + x) * BLK
                pltpu.async_copy(dst1.at[pl.ds(e0, BLK)], didx[x], smi[x])
        return 0
    lax.fori_loop(0, NQ_DEG, quad, 0)
    pltpu.sync_copy(dst1.at[pl.ds(e_base + (NBF_DEG - 1) * BLK, BLK)],
                    didx[0])
    pltpu.sync_copy(ones_v, acc.at[didx[0]], add=True)
    pltpu.sync_copy(dst1.at[pl.ds(e_base + NBF_DEG * BLK, TL_DEG)], didx_t)
    pltpu.sync_copy(ones_v.at[pl.ds(0, TL_DEG)], acc.at[didx_t], add=True)
    plsc.subcore_barrier()

    def _writeback(out):
        for k in range(NZC):
            pltpu.sync_copy(acc.at[pl.ds(t0 + k * BLK, BLK)], zb)
            pltpu.sync_copy(zb, out.at[pl.ds(t0 + k * BLK, BLK)])
        pltpu.sync_copy(acc.at[pl.ds(t0 + NZC * BLK, NZT)],
                        zb.at[pl.ds(0, NZT)])
        pltpu.sync_copy(zb.at[pl.ds(0, NZT)],
                        out.at[pl.ds(t0 + NZC * BLK, NZT)])

    @pl.when(cid == 0)
    def _():
        _writeback(o0)

    @pl.when(cid == 1)
    def _():
        _writeback(o1)


@functools.partial(
    pl.kernel,
    out_type=[jax.ShapeDtypeStruct((NP_D, 16), f32),
              jax.ShapeDtypeStruct((NP_D, 16), f32)],
    mesh=_mesh,
    compiler_params=pltpu.CompilerParams(use_tc_tiling_on_sc=False),
    scratch_types=[
        [pltpu.VMEM((BLK,), i32) for _ in range(3)],
        [pltpu.VMEM((BLK,), i32) for _ in range(3)],
        [pltpu.VMEM((BLK, 16), f32) for _ in range(3)],
        pltpu.VMEM((TL_SEG,), i32),
        pltpu.VMEM((TL_SEG,), i32),
        pltpu.VMEM((TL_SEG, 16), f32),
        pltpu.VMEM_SHARED((NP_D, 16), f32),
        [pltpu.SemaphoreType.DMA for _ in range(3)],
        [pltpu.SemaphoreType.DMA for _ in range(3)],
        [pltpu.SemaphoreType.DMA for _ in range(3)],
        [pltpu.SemaphoreType.DMA for _ in range(3)],
    ],
)
def _seg_kernel(ua, ub, src1, dst1, oa, ob, sidx, didx, rows,
                sidx_t, didx_t, rows_t, acc, smi_s, smi_d, smg, sms):
    cid = lax.axis_index("c")
    sid = lax.axis_index("s")
    t0 = sid * NPT_D

    def run(table, out):
        def init_chunk(k, sz):
            pltpu.sync_copy(table.at[pl.ds(t0 + k * BLK, sz)],
                            rows[0].at[pl.ds(0, sz)])
            pltpu.sync_copy(rows[0].at[pl.ds(0, sz)],
                            acc.at[pl.ds(t0 + k * BLK, sz)])

        for k in range(NZC):
            init_chunk(k, BLK)

        @pl.when(sid < 15)
        def _():
            init_chunk(NZC, NZT)

        @pl.when(sid == 15)
        def _():
            base = 15 * NPT_D + NZC * BLK
            pltpu.sync_copy(table.at[pl.ds(base, 16)],
                            rows[0].at[pl.ds(0, 16)])

            def zfill(i, _):
                rows[0][16 + i, :] = jnp.zeros((16,), f32)
                return 0
            lax.fori_loop(0, NZT - 16, zfill, 0)
            pltpu.sync_copy(rows[0].at[pl.ds(0, NZT)],
                            acc.at[pl.ds(base, NZT)])
        plsc.subcore_barrier()

        e_base = sid * ET_SEG

        for x in range(3):
            pltpu.async_copy(src1.at[pl.ds(e_base + x * BLK, BLK)],
                             sidx[x], smi_s[x])

        def sub_fire(b, x):
            e0 = e_base + b * BLK

            @pl.when(b >= 3)
            def _():
                pltpu.make_async_copy(rows[x], acc.at[didx[x]],
                                      sms[x]).wait()
            pltpu.async_copy(dst1.at[pl.ds(e0, BLK)], didx[x], smi_d[x])
            pltpu.make_async_copy(src1.at[pl.ds(e0, BLK)],
                                  sidx[x], smi_s[x]).wait()
            pltpu.async_copy(table.at[sidx[x]], rows[x], smg[x])

        def sub_drain(b, x):
            e0 = e_base + b * BLK
            pltpu.make_async_copy(table.at[sidx[x]], rows[x],
                                  smg[x]).wait()

            @pl.when(b + 3 < NBF_SEG)
            def _():
                pltpu.async_copy(src1.at[pl.ds(e0 + 3 * BLK, BLK)],
                                 sidx[x], smi_s[x])
            pltpu.make_async_copy(dst1.at[pl.ds(e0, BLK)],
                                  didx[x], smi_d[x]).wait()
            pltpu.async_copy(rows[x], acc.at[didx[x]], sms[x], add=True)

        def triple(k, _):
            for x in range(3):
                sub_fire(3 * k + x, x)
            for x in range(3):
                sub_drain(3 * k + x, x)
            return 0
        lax.fori_loop(0, NBF_SEG // 3, triple, 0)

        et = e_base + NBF_SEG * BLK
        pltpu.sync_copy(src1.at[pl.ds(et, TL_SEG)], sidx_t)
        pltpu.sync_copy(dst1.at[pl.ds(et, TL_SEG)], didx_t)
        pltpu.async_copy(table.at[sidx_t], rows_t, smg[0]).wait()
        pltpu.sync_copy(rows_t, acc.at[didx_t], add=True)

        for x in range(3):
            pltpu.make_async_copy(rows[x], acc.at[didx[x]], sms[x]).wait()
        plsc.subcore_barrier()

        def wb_chunk(k, sz):
            pltpu.sync_copy(acc.at[pl.ds(t0 + k * BLK, sz)],
                            rows[0].at[pl.ds(0, sz)])
            pltpu.sync_copy(rows[0].at[pl.ds(0, sz)],
                            out.at[pl.ds(t0 + k * BLK, sz)])

        for k in range(NZC):
            wb_chunk(k, BLK)
        wb_chunk(NZC, NZT)

    @pl.when(cid == 0)
    def _():
        run(ua, oa)

    @pl.when(cid == 1)
    def _():
        run(ub, ob)


def _col(ref):
    return jnp.sum(ref[...], axis=(0, 1))[:, None]


_SCAL = pl.BlockSpec((1, 1, BN), lambda i: (i, 0, 0))
_ROWS16 = pl.BlockSpec((BN, 16), lambda i: (i, 0))
_ROWS32 = pl.BlockSpec((BN, D_ENC), lambda i: (i, 0))
_W22 = pl.BlockSpec((D_ENC, D_ENC), lambda i: (0, 0))
_B32 = pl.BlockSpec((1, D_ENC), lambda i: (0, 0))


def _enc_prep_body(x_ref, nt_ref, d0_ref, d1_ref, w1_ref, b1_ref, w2_ref,
                   b2_ref, gw1_ref, ua_ref, ub_ref):
    x = x_ref[...]
    za = jnp.maximum(jnp.dot(x, w1_ref[0], preferred_element_type=f32)
                     + b1_ref[0], 0.0)
    za = jnp.maximum(jnp.dot(za, w2_ref[0], preferred_element_type=f32)
                     + b2_ref[0], 0.0)
    zb = jnp.maximum(jnp.dot(x, w1_ref[1], preferred_element_type=f32)
                     + b1_ref[1], 0.0)
    zb = jnp.maximum(jnp.dot(zb, w2_ref[1], preferred_element_type=f32)
                     + b2_ref[1], 0.0)
    h = jnp.where(_col(nt_ref) == 0, za, zb)
    dinv = lax.rsqrt(_col(d0_ref) + _col(d1_ref) + 1.0)
    u = jnp.dot(h, gw1_ref[...], preferred_element_type=f32) * dinv
    ua_ref[...] = u[:, :16]
    ub_ref[...] = u[:, 16:]


def _enc_prep(x, nt50, d050, d150, enc_W1, enc_b1, enc_W2, enc_b2, gcn_W1):
    return pl.pallas_call(
        _enc_prep_body,
        grid=(GRID,),
        in_specs=[
            pl.BlockSpec((BN, D_IN), lambda i: (i, 0)),
            _SCAL, _SCAL, _SCAL,
            pl.BlockSpec((NT, D_IN, H_ENC), lambda i: (0, 0, 0)),
            pl.BlockSpec((NT, H_ENC), lambda i: (0, 0)),
            pl.BlockSpec((NT, H_ENC, D_ENC), lambda i: (0, 0, 0)),
            pl.BlockSpec((NT, D_ENC), lambda i: (0, 0)),
            _W22,
        ],
        out_specs=[_ROWS16, _ROWS16],
        out_shape=[
            jax.ShapeDtypeStruct((N, 16), f32),
            jax.ShapeDtypeStruct((N, 16), f32),
        ],
    )(x, nt50, d050, d150, enc_W1, enc_b1, enc_W2, enc_b2, gcn_W1)


def _mid_body(sa_ref, sb_ref, d0_ref, d1_ref, gw2_ref, b1_ref,
              h1_ref, va_ref, vb_ref):
    dinv = lax.rsqrt(_col(d0_ref) + _col(d1_ref) + 1.0)
    t = jnp.concatenate([sa_ref[...], sb_ref[...]], axis=1)
    h1 = jnp.maximum(t * dinv + b1_ref[...], 0.0)
    h1_ref[...] = h1
    v = jnp.dot(h1, gw2_ref[...], preferred_element_type=f32) * dinv
    va_ref[...] = v[:, :16]
    vb_ref[...] = v[:, 16:]


def _mid(s1a, s1b, d050, d150, gcn_W2, b1r):
    return pl.pallas_call(
        _mid_body,
        grid=(GRID,),
        in_specs=[_ROWS16, _ROWS16, _SCAL, _SCAL, _W22, _B32],
        out_specs=[_ROWS32, _ROWS16, _ROWS16],
        out_shape=[
            jax.ShapeDtypeStruct((N, D_ENC), f32),
            jax.ShapeDtypeStruct((N, 16), f32),
            jax.ShapeDtypeStruct((N, 16), f32),
        ],
    )(s1a, s1b, d050, d150, gcn_W2, b1r)


def _final_body(sa_ref, sb_ref, d0_ref, d1_ref, h1_ref, b2_ref, out_ref):
    dinv = lax.rsqrt(_col(d0_ref) + _col(d1_ref) + 1.0)
    t = jnp.concatenate([sa_ref[...], sb_ref[...]], axis=1)
    h2 = jnp.maximum(t * dinv + b2_ref[...], 0.0)
    out_ref[...] = jnp.concatenate([h1_ref[...], h2], axis=1)


def _final(s2a, s2b, d050, d150, h1, b2r):
    return pl.pallas_call(
        _final_body,
        grid=(GRID,),
        in_specs=[_ROWS16, _ROWS16, _SCAL, _SCAL, _ROWS32, _B32],
        out_specs=pl.BlockSpec((BN, 2 * D_ENC), lambda i: (i, 0)),
        out_shape=jax.ShapeDtypeStruct((N, 2 * D_ENC), f32),
    )(s2a, s2b, d050, d150, h1, b2r)


def kernel(x, edge_index, node_type, enc_W1, enc_b1, enc_W2, enc_b2,
           gcn_W1, gcn_b1, gcn_W2, gcn_b2):
    src1 = edge_index[0]
    dst1 = edge_index[1]

    d0, d1 = _deg_kernel(dst1)
    d0b = lax.optimization_barrier(
        lax.broadcast_in_dim(d0, (1, NP_D), (1,)))
    d1b = lax.optimization_barrier(
        lax.broadcast_in_dim(d1, (1, NP_D), (1,)))
    d050 = d0b[:, :N].reshape(GRID, 1, BN)
    d150 = d1b[:, :N].reshape(GRID, 1, BN)

    nt50 = node_type.reshape(GRID, 1, BN)
    b1r = gcn_b1[None, :]
    b2r = gcn_b2[None, :]

    u1a, u1b = _enc_prep(x, nt50, d050, d150, enc_W1, enc_b1,
                         enc_W2, enc_b2, gcn_W1)
    s1a, s1b = _seg_kernel(u1a, u1b, src1, dst1)
    h1, u2a, u2b = _mid(s1a, s1b, d050, d150, gcn_W2, b1r)
    s2a, s2b = _seg_kernel(u2a, u2b, src1, dst1)
    return _final(s2a, s2b, d050, d150, h1, b2r)
